# baseline (speedup 1.0000x reference)
import jax
import jax.numpy as jnp
from jax import lax
from jax.experimental import pallas as pl
from jax.experimental.pallas import tpu as pltpu
from jax.experimental.pallas import tpu_sc as plsc

N = 10000
E = 320000
D = 128
DH = D // 2
NS = 16
B = 125
W = E // NS // B
CH = 40
NCHUNK = W // CH
NBUF = 4
ROWS_PER_TILE = N // NS
ZROWS = 125


def _body(x2_hbm, col_hbm, row_hbm, out_hbm,
          x_sh, acc_sh, colbuf, rowbuf, msg,
          gsem0, gsem1, gsem2, gsem3, ssem0, ssem1, ssem2, ssem3):
    c = lax.axis_index("c")
    s = lax.axis_index("s")
    r0 = s * ROWS_PER_TILE
    c0 = c * DH

    pltpu.sync_copy(x2_hbm.at[pl.ds(c * N + r0, ROWS_PER_TILE)],
                    x_sh.at[pl.ds(r0, ROWS_PER_TILE)])

    zeros16 = jnp.zeros((16,), jnp.float32)

    def _zero_row(r, carry):
        for k in range(DH // 16):
            msg[0, r, pl.ds(k * 16, 16)] = zeros16
        return carry

    lax.fori_loop(0, ZROWS, _zero_row, 0)
    for b in range(ROWS_PER_TILE // ZROWS):
        pltpu.sync_copy(msg.at[0], acc_sh.at[pl.ds(r0 + b * ZROWS, ZROWS)])

    plsc.subcore_barrier()

    gsems = (gsem0, gsem1, gsem2, gsem3)
    ssems = (ssem0, ssem1, ssem2, ssem3)

    def _start_gather(w, q):
        pltpu.async_copy(x_sh.at[colbuf.at[w]], msg.at[q], gsems[q])

    def _wait_gather(w, q):
        pltpu.make_async_copy(x_sh.at[colbuf.at[w]], msg.at[q],
                              gsems[q]).wait()

    def _start_scatter(w, q):
        pltpu.async_copy(msg.at[q], acc_sh.at[rowbuf.at[w]], ssems[q],
                         add=True)

    def _wait_scatter(w, q):
        pltpu.make_async_copy(msg.at[q], acc_sh.at[rowbuf.at[w]],
                              ssems[q]).wait()

    def _chunk(k, carry):
        w0 = s * W + k * CH
        pltpu.sync_copy(col_hbm.at[pl.ds(w0, CH)], colbuf)
        pltpu.sync_copy(row_hbm.at[pl.ds(w0, CH)], rowbuf)
        for w in range(2):
            _start_gather(w, w)
        for w in range(2):
            _wait_gather(w, w)
            _start_scatter(w, w)
            _start_gather(w + 2, w + 2)

        def _quad(o, carry2):
            wbase = 4 * o + 2
            for i in range(4):
                w = wbase + i
                q = (2 + i) % 4
                qn = i % 4
                _wait_gather(w, q)
                _start_scatter(w, q)
                _wait_scatter(w - 2, qn)
                _start_gather(w + 2, qn)
            return carry2

        lax.fori_loop(0, (CH - 4) // 4, _quad, 0)
        for w in range(CH - 2, CH):
            q = w % 4
            _wait_gather(w, q)
            _start_scatter(w, q)
            _wait_scatter(w - 2, (w - 2) % 4)
        for w in range(CH - 2, CH):
            _wait_scatter(w, w % 4)
        return carry

    lax.fori_loop(0, NCHUNK, _chunk, 0)

    plsc.subcore_barrier()

    pltpu.sync_copy(acc_sh.at[pl.ds(r0, ROWS_PER_TILE)],
                    out_hbm.at[pl.ds(r0, ROWS_PER_TILE), pl.ds(c0, DH)])


@jax.jit
def kernel(x, edge_index):
    x2 = jnp.concatenate([x[:, :DH], x[:, DH:]], axis=0)
    col2 = edge_index[1].reshape(E // B, B)
    row2 = edge_index[0].reshape(E // B, B)

    mesh = plsc.VectorSubcoreMesh(core_axis_name="c", subcore_axis_name="s")
    out = pl.kernel(
        _body,
        out_type=jax.ShapeDtypeStruct((N, D), jnp.float32),
        mesh=mesh,
        scratch_types=[
            pltpu.VMEM_SHARED((N, DH), jnp.float32),
            pltpu.VMEM_SHARED((N, DH), jnp.float32),
            pltpu.VMEM((CH, B), jnp.int32),
            pltpu.VMEM((CH, B), jnp.int32),
            pltpu.VMEM((NBUF, B, DH), jnp.float32),
            pltpu.SemaphoreType.DMA,
            pltpu.SemaphoreType.DMA,
            pltpu.SemaphoreType.DMA,
            pltpu.SemaphoreType.DMA,
            pltpu.SemaphoreType.DMA,
            pltpu.SemaphoreType.DMA,
            pltpu.SemaphoreType.DMA,
            pltpu.SemaphoreType.DMA,
        ],
        compiler_params=pltpu.CompilerParams(use_tc_tiling_on_sc=False),
    )(x2, col2, row2)
    return out
